# baseline (speedup 1.0000x reference)
import functools

import jax
import jax.numpy as jnp
from jax import lax
from jax.experimental import pallas as pl
from jax.experimental.pallas import tpu as pltpu
from jax.experimental.pallas import tpu_sc as plsc

_TOK_BLOCK = 1024


def _argmin_body(xsq_ref, x_ref, embt_ref, esq_ref, idx_ref):
    ncode = embt_ref.shape[1]
    nch = 4
    chw = ncode // nch
    x2 = x_ref[...] * 2.0
    d = lax.dot_general(x2, embt_ref[...], (((1,), (0,)), ((), ())),
                        preferred_element_type=jnp.float32)
    t = (xsq_ref[...] - d) + esq_ref[...]
    ii = lax.broadcasted_iota(jnp.int32, (1, chw), 1)
    ms, is_ = [], []
    for k in range(nch):
        tc = t[:, k * chw:(k + 1) * chw]
        m = jnp.min(tc, axis=1)
        cand = jnp.where(tc == m[:, None], ii, jnp.int32(ncode))
        ms.append(m)
        is_.append(jnp.min(cand, axis=1) + jnp.int32(k * chw))
    take1 = ms[1] < ms[0]
    m_l = jnp.where(take1, ms[1], ms[0])
    i_l = jnp.where(take1, is_[1], is_[0])
    take3 = ms[3] < ms[2]
    m_r = jnp.where(take3, ms[3], ms[2])
    i_r = jnp.where(take3, is_[3], is_[2])
    bl = lax.bitcast_convert_type(m_l, jnp.int32)
    br = lax.bitcast_convert_type(m_r, jnp.int32)
    hi_l = lax.shift_right_logical(bl, 16)
    hi_r = lax.shift_right_logical(br, 16)
    b15_l = lax.shift_right_logical(bl, 15) & 1
    take_l = (hi_l < hi_r) | ((hi_l == hi_r) & (b15_l == 0))
    idx_ref[...] = jnp.where(take_l, i_l, i_r)


def _argmin_call(xsq, flat, embt, esq):
    n_tok, d_model = flat.shape
    ncode = embt.shape[1]
    t = _TOK_BLOCK
    grid = n_tok // t
    return pl.pallas_call(
        _argmin_body,
        grid=(grid,),
        in_specs=[
            pl.BlockSpec((t, 1), lambda i: (i, 0)),
            pl.BlockSpec((t, d_model), lambda i: (i, 0)),
            pl.BlockSpec((d_model, ncode), lambda i: (0, 0)),
            pl.BlockSpec((1, ncode), lambda i: (0, 0)),
        ],
        out_specs=pl.BlockSpec((t,), lambda i: (i,)),
        out_shape=jax.ShapeDtypeStruct((n_tok,), jnp.int32),
    )(xsq, flat, embt, esq)


def _sc_gather(embp, idx3, flat):
    info = plsc.get_sparse_core_info()
    nc, ns, lanes = info.num_cores, info.num_subcores, info.num_lanes
    nw = nc * ns
    b, d_model = flat.shape
    rowp = embp.shape[1]
    bpw = b // nw
    nch = idx3.shape[1]
    chw = idx3.shape[2]
    mesh = plsc.VectorSubcoreMesh(core_axis_name="c", subcore_axis_name="s")

    @functools.partial(
        pl.kernel,
        mesh=mesh,
        out_type=[
            jax.ShapeDtypeStruct((b, d_model), jnp.float32),
            jax.ShapeDtypeStruct((nw, lanes), jnp.float32),
        ],
        scratch_types=[
            pltpu.VMEM((nch, chw), jnp.int32),
            pltpu.VMEM((2, chw, rowp), jnp.float32),
            pltpu.VMEM((bpw, d_model), jnp.float32),
            pltpu.VMEM((lanes,), jnp.float32),
            pltpu.SemaphoreType.DMA,
            pltpu.SemaphoreType.DMA,
        ],
    )
    def body(emb_hbm, idx_hbm, x_hbm, out_hbm, loss_hbm,
             idx_v, rows_v, x_v, loss_v, sem0, sem1):
        wid = lax.axis_index("s") * nc + lax.axis_index("c")
        base = wid * bpw
        sems = (sem0, sem1)
        pltpu.sync_copy(idx_hbm.at[wid], idx_v)
        pltpu.sync_copy(x_hbm.at[pl.ds(base, bpw)], x_v)

        def fire(ch):
            return pltpu.async_copy(emb_hbm.at[idx_v.at[ch]],
                                    rows_v.at[ch % 2], sems[ch % 2])

        nvec = d_model // lanes
        zero = jnp.zeros((lanes,), jnp.float32)
        acc = (zero,) * nvec
        handles = {0: fire(0)}
        for ch in range(nch):
            if ch + 1 < nch:
                handles[ch + 1] = fire(ch + 1)
            handles[ch].wait()
            buf = ch % 2

            def tok_body(r, a, _ch=ch, _buf=buf):
                new = []
                for j in range(nvec):
                    e = rows_v[_buf, r, pl.ds(j * lanes, lanes)]
                    xv = x_v[_ch * chw + r, pl.ds(j * lanes, lanes)]
                    diff = e - xv
                    x_v[_ch * chw + r, pl.ds(j * lanes, lanes)] = xv + diff
                    new.append(a[j] + diff * diff)
                return tuple(new)

            acc = lax.fori_loop(0, chw, tok_body, acc)
        tot = acc[0]
        for j in range(1, nvec):
            tot = tot + acc[j]
        loss_v[...] = tot
        pltpu.sync_copy(x_v, out_hbm.at[pl.ds(base, bpw)])
        pltpu.sync_copy(loss_v, loss_hbm.at[wid])

    return body(embp, idx3, flat)


def kernel(x, emb):
    d_model = x.shape[-1]
    flat = x.reshape(-1, d_model)
    n_tok = flat.shape[0]
    xsq = jnp.sum(flat ** 2, axis=1, keepdims=True)
    esq = jnp.sum(emb ** 2, axis=1)[None, :]
    embt = emb.T
    idx_flat = _argmin_call(xsq, flat, embt, esq)

    nw = 32
    chw = 128
    idx3 = idx_flat.reshape(nw, n_tok // (nw * chw), chw)
    embp = jnp.pad(emb, ((0, 0), (0, 128 - d_model)))
    out_flat, loss_parts = _sc_gather(embp, idx3, flat)
    loss = jnp.sum(loss_parts) / (n_tok * d_model)
    return out_flat.reshape(x.shape), idx_flat.reshape(x.shape[:2]), loss
